# Initial kernel scaffold; baseline (speedup 1.0000x reference)
import functools

import jax
import jax.numpy as jnp
from jax import lax
from jax.experimental import pallas as pl
from jax.experimental.pallas import tpu as pltpu
from jax.experimental.pallas import tpu_sc as plsc

HIDDEN = 128
SEQ = 200
EPS = 1e-12

NC = 2
NS = 16
NW = NC * NS

CHUNK = 128
POS_BUF = 320
LANES = 16
NVEC = HIDDEN // LANES


def _rsqrt(v):
    i = plsc.bitcast(v, jnp.int32)
    i = jnp.int32(0x5F3759DF) - lax.shift_right_logical(i, jnp.int32(1))
    y = plsc.bitcast(i, jnp.float32)
    for _ in range(3):
        y = y * (1.5 - 0.5 * v * y * y)
    return y


def kernel(input_ids, word_emb, pos_emb, gamma, beta):
    B, L = input_ids.shape
    N = B * L
    ids = input_ids.reshape(-1).astype(jnp.int32)
    rows_per_w = N // NW
    n_chunks = rows_per_w // CHUNK

    mesh = plsc.VectorSubcoreMesh(core_axis_name="c", subcore_axis_name="s")

    @functools.partial(
        pl.kernel,
        mesh=mesh,
        out_type=jax.ShapeDtypeStruct((N, HIDDEN), jnp.float32),
        scratch_types=[
            pltpu.VMEM((CHUNK,), jnp.int32),
            pltpu.VMEM((CHUNK, HIDDEN), jnp.float32),
            pltpu.VMEM((POS_BUF, HIDDEN), jnp.float32),
            pltpu.VMEM((HIDDEN,), jnp.float32),
            pltpu.VMEM((HIDDEN,), jnp.float32),
            pltpu.SemaphoreType.DMA,
        ],
    )
    def emb_kernel(ids_hbm, wemb_hbm, pemb_hbm, gamma_hbm, beta_hbm, out_hbm,
                   idx_v, rows_v, pos_v, g_v, b_v, sem):
        wid = lax.axis_index("s") * NC + lax.axis_index("c")

        pltpu.sync_copy(pemb_hbm.at[pl.ds(0, SEQ)], pos_v.at[pl.ds(0, SEQ)])
        pltpu.sync_copy(pemb_hbm.at[pl.ds(0, POS_BUF - SEQ)],
                        pos_v.at[pl.ds(SEQ, POS_BUF - SEQ)])
        pltpu.sync_copy(gamma_hbm, g_v)
        pltpu.sync_copy(beta_hbm, b_v)

        def chunk_body(c, carry):
            base = wid * rows_per_w + c * CHUNK
            off = lax.rem(c * CHUNK, SEQ)
            pltpu.sync_copy(ids_hbm.at[pl.ds(base, CHUNK)], idx_v)
            pltpu.async_copy(wemb_hbm.at[idx_v], rows_v, sem).wait()

            def row_body(i, rcarry):
                x = [rows_v[i, pl.ds(LANES * j, LANES)] +
                     pos_v[off + i, pl.ds(LANES * j, LANES)]
                     for j in range(NVEC)]
                s = ((x[0] + x[1]) + (x[2] + x[3])) + ((x[4] + x[5]) + (x[6] + x[7]))
                q = ((x[0] * x[0] + x[1] * x[1]) + (x[2] * x[2] + x[3] * x[3])) + \
                    ((x[4] * x[4] + x[5] * x[5]) + (x[6] * x[6] + x[7] * x[7]))
                total = jnp.sum(s)
                total_sq = jnp.sum(q)
                mean = total * (1.0 / HIDDEN)
                var = total_sq * (1.0 / HIDDEN) - mean * mean
                mean_v = jnp.broadcast_to(mean, (LANES,))
                rstd_v = _rsqrt(jnp.broadcast_to(var + EPS, (LANES,)))
                for j in range(NVEC):
                    g = g_v[pl.ds(LANES * j, LANES)]
                    b = b_v[pl.ds(LANES * j, LANES)]
                    rows_v[i, pl.ds(LANES * j, LANES)] = \
                        (x[j] - mean_v) * rstd_v * g + b
                return rcarry

            lax.fori_loop(0, CHUNK, row_body, 0)
            pltpu.sync_copy(rows_v, out_hbm.at[pl.ds(base, CHUNK)])
            return carry

        lax.fori_loop(0, n_chunks, chunk_body, 0)

    out = emb_kernel(ids, word_emb, pos_emb, gamma, beta)
    return out.reshape(B, L, HIDDEN)


# speedup vs baseline: 2.1542x; 2.1542x over previous
import functools

import jax
import jax.numpy as jnp
from jax import lax
from jax.experimental import pallas as pl
from jax.experimental.pallas import tpu as pltpu
from jax.experimental.pallas import tpu_sc as plsc

HIDDEN = 128
SEQ = 200
EPS = 1e-12

NC = 2
NS = 16
NW = NC * NS

CHUNK = 128
POS_BUF = 320
LANES = 16
NVEC = HIDDEN // LANES


def _rsqrt(v):
    i = plsc.bitcast(v, jnp.int32)
    i = jnp.int32(0x5F3759DF) - lax.shift_right_logical(i, jnp.int32(1))
    y = plsc.bitcast(i, jnp.float32)
    for _ in range(3):
        y = y * (1.5 - 0.5 * v * y * y)
    return y


def kernel(input_ids, word_emb, pos_emb, gamma, beta):
    B, L = input_ids.shape
    N = B * L
    ids = input_ids.reshape(-1).astype(jnp.int32)
    rows_per_w = N // NW
    n_chunks = rows_per_w // CHUNK

    mesh = plsc.VectorSubcoreMesh(core_axis_name="c", subcore_axis_name="s")

    @functools.partial(
        pl.kernel,
        mesh=mesh,
        out_type=jax.ShapeDtypeStruct((N, HIDDEN), jnp.float32),
        compiler_params=pltpu.CompilerParams(needs_layout_passes=False),
        scratch_types=[
            pltpu.VMEM((CHUNK,), jnp.int32),
            pltpu.VMEM((CHUNK, HIDDEN), jnp.float32),
            pltpu.VMEM((POS_BUF, HIDDEN), jnp.float32),
            pltpu.VMEM((HIDDEN,), jnp.float32),
            pltpu.VMEM((HIDDEN,), jnp.float32),
            pltpu.SemaphoreType.DMA,
        ],
    )
    def emb_kernel(ids_hbm, wemb_hbm, pemb_hbm, gamma_hbm, beta_hbm, out_hbm,
                   idx_v, rows_v, pos_v, g_v, b_v, sem):
        wid = lax.axis_index("s") * NC + lax.axis_index("c")

        pltpu.sync_copy(pemb_hbm.at[pl.ds(0, SEQ)], pos_v.at[pl.ds(0, SEQ)])
        pltpu.sync_copy(pemb_hbm.at[pl.ds(0, POS_BUF - SEQ)],
                        pos_v.at[pl.ds(SEQ, POS_BUF - SEQ)])
        pltpu.sync_copy(gamma_hbm, g_v)
        pltpu.sync_copy(beta_hbm, b_v)

        def chunk_body(c, carry):
            base = wid * rows_per_w + c * CHUNK
            off = lax.rem(c * CHUNK, SEQ)
            pltpu.sync_copy(ids_hbm.at[pl.ds(base, CHUNK)], idx_v)
            pltpu.async_copy(wemb_hbm.at[idx_v], rows_v, sem).wait()

            def row_body(i, rcarry):
                x = [rows_v[i, pl.ds(LANES * j, LANES)] +
                     pos_v[off + i, pl.ds(LANES * j, LANES)]
                     for j in range(NVEC)]
                s = ((x[0] + x[1]) + (x[2] + x[3])) + ((x[4] + x[5]) + (x[6] + x[7]))
                q = ((x[0] * x[0] + x[1] * x[1]) + (x[2] * x[2] + x[3] * x[3])) + \
                    ((x[4] * x[4] + x[5] * x[5]) + (x[6] * x[6] + x[7] * x[7]))
                total = jnp.sum(s)
                total_sq = jnp.sum(q)
                mean = total * (1.0 / HIDDEN)
                var = total_sq * (1.0 / HIDDEN) - mean * mean
                mean_v = jnp.broadcast_to(mean, (LANES,))
                rstd_v = _rsqrt(jnp.broadcast_to(var + EPS, (LANES,)))
                for j in range(NVEC):
                    g = g_v[pl.ds(LANES * j, LANES)]
                    b = b_v[pl.ds(LANES * j, LANES)]
                    rows_v[i, pl.ds(LANES * j, LANES)] = \
                        (x[j] - mean_v) * rstd_v * g + b
                return rcarry

            lax.fori_loop(0, CHUNK, row_body, 0)
            pltpu.sync_copy(rows_v, out_hbm.at[pl.ds(base, CHUNK)])
            return carry

        lax.fori_loop(0, n_chunks, chunk_body, 0)

    out = emb_kernel(ids, word_emb, pos_emb, gamma, beta)
    return out.reshape(B, L, HIDDEN)


# speedup vs baseline: 2.3490x; 1.0904x over previous
import functools

import jax
import jax.numpy as jnp
from jax import lax
from jax.experimental import pallas as pl
from jax.experimental.pallas import tpu as pltpu
from jax.experimental.pallas import tpu_sc as plsc

HIDDEN = 128
SEQ = 200
EPS = 1e-12

NC = 2
NS = 16
NW = NC * NS

CHUNK = 128
POS_BUF = 320
LANES = 16
NVEC = HIDDEN // LANES


def _rsqrt(v):
    i = plsc.bitcast(v, jnp.int32)
    i = jnp.int32(0x5F3759DF) - lax.shift_right_logical(i, jnp.int32(1))
    y = plsc.bitcast(i, jnp.float32)
    for _ in range(3):
        y = y * (1.5 - 0.5 * v * y * y)
    return y


def kernel(input_ids, word_emb, pos_emb, gamma, beta):
    B, L = input_ids.shape
    N = B * L
    rows_per_w = N // NW
    n_chunks = rows_per_w // CHUNK
    ids = input_ids.reshape(NW, n_chunks, CHUNK).astype(jnp.int32)

    mesh = plsc.VectorSubcoreMesh(core_axis_name="c", subcore_axis_name="s")

    @functools.partial(
        pl.kernel,
        mesh=mesh,
        out_type=jax.ShapeDtypeStruct((N, HIDDEN), jnp.float32),
        compiler_params=pltpu.CompilerParams(needs_layout_passes=False),
        scratch_types=[
            pltpu.VMEM((n_chunks, CHUNK), jnp.int32),
            pltpu.VMEM((CHUNK, HIDDEN), jnp.float32),
            pltpu.VMEM((POS_BUF, HIDDEN), jnp.float32),
            pltpu.VMEM((HIDDEN,), jnp.float32),
            pltpu.VMEM((HIDDEN,), jnp.float32),
            pltpu.SemaphoreType.DMA,
        ],
    )
    def emb_kernel(ids_hbm, wemb_hbm, pemb_hbm, gamma_hbm, beta_hbm, out_hbm,
                   idx_all, rows_v, pos_v, g_v, b_v, sem):
        wid = lax.axis_index("s") * NC + lax.axis_index("c")

        pltpu.sync_copy(ids_hbm.at[wid], idx_all)
        pltpu.sync_copy(pemb_hbm.at[pl.ds(0, SEQ)], pos_v.at[pl.ds(0, SEQ)])
        pltpu.sync_copy(pemb_hbm.at[pl.ds(0, POS_BUF - SEQ)],
                        pos_v.at[pl.ds(SEQ, POS_BUF - SEQ)])
        pltpu.sync_copy(gamma_hbm, g_v)
        pltpu.sync_copy(beta_hbm, b_v)

        lane = lax.iota(jnp.int32, LANES)
        perms = [lane ^ k for k in (1, 2, 4, 8)]

        def chunk_body(c, carry):
            base = wid * rows_per_w + c * CHUNK
            off = lax.rem(c * CHUNK, SEQ)
            pltpu.async_copy(wemb_hbm.at[idx_all.at[c]], rows_v, sem).wait()

            def row_body(i, rcarry):
                x = [rows_v[i, pl.ds(LANES * j, LANES)] +
                     pos_v[off + i, pl.ds(LANES * j, LANES)]
                     for j in range(NVEC)]
                s = ((x[0] + x[1]) + (x[2] + x[3])) + ((x[4] + x[5]) + (x[6] + x[7]))
                q = ((x[0] * x[0] + x[1] * x[1]) + (x[2] * x[2] + x[3] * x[3])) + \
                    ((x[4] * x[4] + x[5] * x[5]) + (x[6] * x[6] + x[7] * x[7]))
                for p in perms:
                    s = s + s.at[p].get(mode="promise_in_bounds")
                    q = q + q.at[p].get(mode="promise_in_bounds")
                mean_v = s * (1.0 / HIDDEN)
                var_v = q * (1.0 / HIDDEN) - mean_v * mean_v
                rstd_v = _rsqrt(var_v + EPS)
                for j in range(NVEC):
                    g = g_v[pl.ds(LANES * j, LANES)]
                    b = b_v[pl.ds(LANES * j, LANES)]
                    rows_v[i, pl.ds(LANES * j, LANES)] = \
                        (x[j] - mean_v) * rstd_v * g + b
                return rcarry

            lax.fori_loop(0, CHUNK, row_body, 0)
            pltpu.sync_copy(rows_v, out_hbm.at[pl.ds(base, CHUNK)])
            return carry

        lax.fori_loop(0, n_chunks, chunk_body, 0)

    out = emb_kernel(ids, word_emb, pos_emb, gamma, beta)
    return out.reshape(B, L, HIDDEN)


# speedup vs baseline: 4.1938x; 1.7854x over previous
import functools

import jax
import jax.numpy as jnp
from jax import lax
from jax.experimental import pallas as pl
from jax.experimental.pallas import tpu as pltpu
from jax.experimental.pallas import tpu_sc as plsc

HIDDEN = 128
SEQ = 200
EPS = 1e-12

NC = 2
NS = 16
NW = NC * NS

CHUNK = 128
POS_BUF = 320
LANES = 16
NVEC = HIDDEN // LANES


def _rsqrt(v):
    i = plsc.bitcast(v, jnp.int32)
    i = jnp.int32(0x5F3759DF) - lax.shift_right_logical(i, jnp.int32(1))
    y = plsc.bitcast(i, jnp.float32)
    for _ in range(3):
        y = y * (1.5 - 0.5 * v * y * y)
    return y


def kernel(input_ids, word_emb, pos_emb, gamma, beta):
    B, L = input_ids.shape
    N = B * L
    rows_per_w = N // NW
    n_chunks = rows_per_w // CHUNK
    ids = input_ids.reshape(NW, n_chunks, CHUNK).astype(jnp.int32)

    mesh = plsc.VectorSubcoreMesh(core_axis_name="c", subcore_axis_name="s")

    @functools.partial(
        pl.kernel,
        mesh=mesh,
        out_type=jax.ShapeDtypeStruct((N, HIDDEN), jnp.float32),
        compiler_params=pltpu.CompilerParams(needs_layout_passes=False),
        scratch_types=[
            pltpu.VMEM((n_chunks, CHUNK), jnp.int32),
            pltpu.VMEM((CHUNK, HIDDEN), jnp.float32),
            pltpu.VMEM((POS_BUF, HIDDEN), jnp.float32),
            pltpu.VMEM((HIDDEN,), jnp.float32),
            pltpu.VMEM((HIDDEN,), jnp.float32),
            pltpu.SemaphoreType.DMA,
        ],
    )
    def emb_kernel(ids_hbm, wemb_hbm, pemb_hbm, gamma_hbm, beta_hbm, out_hbm,
                   idx_all, rows_v, pos_v, g_v, b_v, sem):
        wid = lax.axis_index("s") * NC + lax.axis_index("c")

        pltpu.sync_copy(ids_hbm.at[wid], idx_all)
        pltpu.sync_copy(pemb_hbm.at[pl.ds(0, SEQ)], pos_v.at[pl.ds(0, SEQ)])
        pltpu.sync_copy(pemb_hbm.at[pl.ds(0, POS_BUF - SEQ)],
                        pos_v.at[pl.ds(SEQ, POS_BUF - SEQ)])
        pltpu.sync_copy(gamma_hbm, g_v)
        pltpu.sync_copy(beta_hbm, b_v)

        lane = lax.iota(jnp.int32, LANES)
        perms = [lane ^ k for k in (1, 2, 4, 8)]

        def chunk_body(c, carry):
            base = wid * rows_per_w + c * CHUNK
            off = lax.rem(c * CHUNK, SEQ)
            pltpu.async_copy(wemb_hbm.at[idx_all.at[c]], rows_v, sem).wait()

            @plsc.parallel_loop(0, CHUNK, unroll=4)
            def row_body(i):
                x = [rows_v[i, pl.ds(LANES * j, LANES)] +
                     pos_v[off + i, pl.ds(LANES * j, LANES)]
                     for j in range(NVEC)]
                s = ((x[0] + x[1]) + (x[2] + x[3])) + ((x[4] + x[5]) + (x[6] + x[7]))
                q = ((x[0] * x[0] + x[1] * x[1]) + (x[2] * x[2] + x[3] * x[3])) + \
                    ((x[4] * x[4] + x[5] * x[5]) + (x[6] * x[6] + x[7] * x[7]))
                for p in perms:
                    s = s + s.at[p].get(mode="promise_in_bounds")
                    q = q + q.at[p].get(mode="promise_in_bounds")
                mean_v = s * (1.0 / HIDDEN)
                var_v = q * (1.0 / HIDDEN) - mean_v * mean_v
                rstd_v = _rsqrt(var_v + EPS)
                for j in range(NVEC):
                    g = g_v[pl.ds(LANES * j, LANES)]
                    b = b_v[pl.ds(LANES * j, LANES)]
                    rows_v[i, pl.ds(LANES * j, LANES)] = \
                        (x[j] - mean_v) * rstd_v * g + b

            pltpu.sync_copy(rows_v, out_hbm.at[pl.ds(base, CHUNK)])
            return carry

        lax.fori_loop(0, n_chunks, chunk_body, 0)

    out = emb_kernel(ids, word_emb, pos_emb, gamma, beta)
    return out.reshape(B, L, HIDDEN)
